# baseline (speedup 1.0000x reference)
import functools

import jax
import jax.numpy as jnp
from jax import lax
from jax.experimental import pallas as pl
from jax.experimental.pallas import tpu as pltpu
from jax.experimental.pallas import tpu_sc as plsc

N = 10000
E = 320000
D = 128

NC = 2
NS = 16
NW = NC * NS
NPAD = 10240
RPT = NPAD // NS
CH = 64
NCH = RPT // CH
EPWP = 10240
EP = EPWP * NW

_mesh = plsc.VectorSubcoreMesh(core_axis_name="c", subcore_axis_name="s")


def _make_sc_agg(batch, with_counts):
    niter = EPWP // batch
    assert niter % 2 == 0 and niter >= 4

    out_type = [jax.ShapeDtypeStruct((NC, NPAD, D), jnp.float32)]
    if with_counts:
        out_type.append(jax.ShapeDtypeStruct((NC, NPAD, 16), jnp.float32))

    scratch = [
        pltpu.VMEM((batch,), jnp.int32),
        pltpu.VMEM((batch,), jnp.int32),
        pltpu.VMEM((batch,), jnp.int32),
        pltpu.VMEM((batch,), jnp.int32),
        pltpu.VMEM((batch, D), jnp.float32),
        pltpu.VMEM((batch, D), jnp.float32),
    ]
    if with_counts:
        scratch.append(pltpu.VMEM((batch, 16), jnp.float32))
    scratch.append(pltpu.VMEM((CH, D), jnp.float32))
    if with_counts:
        scratch.append(pltpu.VMEM((CH, 16), jnp.float32))
    scratch.append(pltpu.VMEM_SHARED((NPAD, D), jnp.float32))
    if with_counts:
        scratch.append(pltpu.VMEM_SHARED((NPAD, 16), jnp.float32))
    nsem = 10 if with_counts else 8
    scratch += [pltpu.SemaphoreType.DMA] * nsem

    @functools.partial(
        pl.kernel,
        mesh=_mesh,
        compiler_params=pltpu.CompilerParams(use_tc_tiling_on_sc=False),
        out_type=out_type,
        scratch_types=scratch,
    )
    def sc_agg(*refs):
        it = iter(refs)
        t_hbm = next(it)
        src_hbm = next(it)
        dst_hbm = next(it)
        zrow_hbm = next(it)
        zcnt_hbm = next(it) if with_counts else None
        ones_hbm = next(it) if with_counts else None
        pacc_hbm = next(it)
        pcnt_hbm = next(it) if with_counts else None
        SRC = (next(it), next(it))
        DST = (next(it), next(it))
        ROWS = (next(it), next(it))
        ones_v = next(it) if with_counts else None
        stage_v = next(it)
        stagec_v = next(it) if with_counts else None
        acc_s = next(it)
        cnt_s = next(it) if with_counts else None
        GS = (next(it), next(it))
        RS = (next(it), next(it))
        CS = (next(it), next(it)) if with_counts else None
        ISS = (next(it), next(it))
        IDS = (next(it), next(it))

        c = lax.axis_index("c")
        s = lax.axis_index("s")
        wid = s * NC + c

        def off(i):
            return pl.multiple_of(wid * EPWP + i * batch, 8)

        def idx_src(i, p):
            return pltpu.make_async_copy(
                src_hbm.at[pl.ds(off(i), batch)], SRC[p], ISS[p])

        def idx_dst(i, p):
            return pltpu.make_async_copy(
                dst_hbm.at[pl.ds(off(i), batch)], DST[p], IDS[p])

        def gath(p):
            return pltpu.make_async_copy(t_hbm.at[SRC[p]], ROWS[p], GS[p])

        def srow(p):
            return pltpu.make_async_copy(ROWS[p], acc_s.at[DST[p]], RS[p])

        def scnt(p):
            return pltpu.make_async_copy(ones_v, cnt_s.at[DST[p]], CS[p])

        pltpu.sync_copy(zrow_hbm, stage_v)
        if with_counts:
            pltpu.sync_copy(zcnt_hbm, stagec_v)
            pltpu.sync_copy(ones_hbm, ones_v)

        def zbody(j, carry):
            o = pl.multiple_of(s * RPT + j * CH, 8)
            pltpu.sync_copy(stage_v, acc_s.at[pl.ds(o, CH)])
            if with_counts:
                pltpu.sync_copy(stagec_v, cnt_s.at[pl.ds(o, CH)])
            return carry

        lax.fori_loop(0, NCH, zbody, 0)
        plsc.subcore_barrier()

        def step(i, p, wait_prev, next_gather, next2_idx):
            q = 1 - p
            gath(p).wait()
            idx_dst(i, p).wait()
            srow(p).start(add=True)
            if with_counts:
                scnt(p).start(add=True)
            if wait_prev:
                srow(q).wait()
                if with_counts:
                    scnt(q).wait()
            if next_gather:
                idx_src(i + 1, q).wait()
                gath(q).start()
                idx_dst(i + 1, q).start()
            if next2_idx:
                idx_src(i + 2, p).start()

        idx_src(0, 0).start()
        idx_dst(0, 0).start()
        idx_src(1, 1).start()
        idx_src(0, 0).wait()
        gath(0).start()

        step(0, 0, False, True, True)
        step(1, 1, True, True, True)

        def pair(k, carry):
            step(2 * k, 0, True, True, True)
            step(2 * k + 1, 1, True, True, True)
            return carry

        lax.fori_loop(1, niter // 2 - 1, pair, 0)
        step(niter - 2, 0, True, True, False)
        step(niter - 1, 1, True, False, False)
        srow(1).wait()
        if with_counts:
            scnt(1).wait()
        plsc.subcore_barrier()

        def wbody(j, carry):
            o = pl.multiple_of(s * RPT + j * CH, 8)
            pltpu.sync_copy(acc_s.at[pl.ds(o, CH)], stage_v)
            pltpu.sync_copy(stage_v, pacc_hbm.at[c, pl.ds(o, CH)])
            if with_counts:
                pltpu.sync_copy(cnt_s.at[pl.ds(o, CH)], stagec_v)
                pltpu.sync_copy(stagec_v, pcnt_hbm.at[c, pl.ds(o, CH)])
            return carry

        lax.fori_loop(0, NCH, wbody, 0)

    return sc_agg


_sc_agg_cnt = _make_sc_agg(80, True)
_sc_agg_nocnt = _make_sc_agg(128, False)


_BLK = 1000
_GRID = N // _BLK


def _dot_t(a, w):
    return lax.dot_general(a, w, (((1,), (1,)), ((), ())),
                           preferred_element_type=jnp.float32)


def _tc_in_body(x_ref, wl_ref, wr_ref, b_ref, t_ref, r_ref):
    xb = x_ref[...]
    t_ref[...] = _dot_t(xb, wl_ref[...])
    r_ref[...] = _dot_t(xb, wr_ref[...]) + b_ref[...]


def _tc_in(x, W_l, W_r, b):
    w_spec = pl.BlockSpec((D, D), lambda i: (0, 0))
    return pl.pallas_call(
        _tc_in_body,
        grid=(_GRID,),
        in_specs=[pl.BlockSpec((_BLK, D), lambda i: (i, 0)), w_spec, w_spec,
                  pl.BlockSpec((1, D), lambda i: (0, 0))],
        out_specs=[pl.BlockSpec((_BLK, D), lambda i: (i, 0))] * 2,
        out_shape=[jax.ShapeDtypeStruct((N, D), jnp.float32)] * 2,
    )(x, W_l, W_r, b.reshape(1, D))


def _tc_mid_body(pacc_ref, pcnt_ref, r_ref, wl_ref, wr_ref, b_ref,
                 t_ref, r2_ref):
    agg = pacc_ref[0] + pacc_ref[1]
    cnt = pcnt_ref[0, :, 0:1] + pcnt_ref[1, :, 0:1]
    mean = agg / jnp.maximum(cnt, 1.0)
    h = jnp.maximum(mean + r_ref[...], 0.0)
    t_ref[...] = _dot_t(h, wl_ref[...])
    r2_ref[...] = _dot_t(h, wr_ref[...]) + b_ref[...]


def _tc_mid(pacc, pcnt, r1, W_l, W_r, b):
    w_spec = pl.BlockSpec((D, D), lambda i: (0, 0))
    return pl.pallas_call(
        _tc_mid_body,
        grid=(_GRID,),
        in_specs=[pl.BlockSpec((NC, _BLK, D), lambda i: (0, i, 0)),
                  pl.BlockSpec((NC, _BLK, 16), lambda i: (0, i, 0)),
                  pl.BlockSpec((_BLK, D), lambda i: (i, 0)),
                  w_spec, w_spec,
                  pl.BlockSpec((1, D), lambda i: (0, 0))],
        out_specs=[pl.BlockSpec((_BLK, D), lambda i: (i, 0))] * 2,
        out_shape=[jax.ShapeDtypeStruct((N, D), jnp.float32)] * 2,
    )(pacc, pcnt, r1, W_l, W_r, b.reshape(1, D))


def _tc_out_body(pacc_ref, pcnt_ref, r_ref, o_ref):
    agg = pacc_ref[0] + pacc_ref[1]
    cnt = pcnt_ref[0, :, 0:1] + pcnt_ref[1, :, 0:1]
    o_ref[...] = agg / jnp.maximum(cnt, 1.0) + r_ref[...]


def _tc_out(pacc, pcnt, r2):
    return pl.pallas_call(
        _tc_out_body,
        grid=(_GRID,),
        in_specs=[pl.BlockSpec((NC, _BLK, D), lambda i: (0, i, 0)),
                  pl.BlockSpec((NC, _BLK, 16), lambda i: (0, i, 0)),
                  pl.BlockSpec((_BLK, D), lambda i: (i, 0))],
        out_specs=pl.BlockSpec((_BLK, D), lambda i: (i, 0)),
        out_shape=jax.ShapeDtypeStruct((N, D), jnp.float32),
    )(pacc, pcnt, r2)


def kernel(x, edge_index, W1_l, b1_l, W1_r, W2_l, b2_l, W2_r):
    src = edge_index[0].astype(jnp.int32)
    dst = edge_index[1].astype(jnp.int32)
    pad = EP - E
    src = jnp.concatenate([src, jnp.zeros((pad,), jnp.int32)])
    dst = jnp.concatenate([dst, jnp.full((pad,), NPAD - 1, jnp.int32)])
    zrow = jnp.zeros((CH, D), jnp.float32)
    zcnt = jnp.zeros((CH, 16), jnp.float32)
    ones = jnp.ones((80, 16), jnp.float32)

    t1, r1 = _tc_in(x, W1_l, W1_r, b1_l)
    pacc1, pcnt1 = _sc_agg_cnt(t1, src, dst, zrow, zcnt, ones)
    t2, r2 = _tc_mid(pacc1, pcnt1, r1, W2_l, W2_r, b2_l)
    (pacc2,) = _sc_agg_nocnt(t2, src, dst, zrow)
    return _tc_out(pacc2, pcnt1, r2)


# speedup vs baseline: 1.1331x; 1.1331x over previous
import functools

import jax
import jax.numpy as jnp
from jax import lax
from jax.experimental import pallas as pl
from jax.experimental.pallas import tpu as pltpu
from jax.experimental.pallas import tpu_sc as plsc

N = 10000
E = 320000
D = 128

NC = 2
NS = 16
NW = NC * NS
NPAD = 10240
RPT = NPAD // NS
CH = 64
NCH = RPT // CH
EPWP = 10240
EP = EPWP * NW

_mesh = plsc.VectorSubcoreMesh(core_axis_name="c", subcore_axis_name="s")


def _make_sc_agg(batch, with_counts):
    niter = EPWP // batch
    assert niter % 2 == 0 and niter >= 4

    out_type = [jax.ShapeDtypeStruct((NC, NPAD, D), jnp.float32)]
    if with_counts:
        out_type.append(jax.ShapeDtypeStruct((NC, NPAD, 16), jnp.float32))

    scratch = [
        pltpu.VMEM((batch,), jnp.int32),
        pltpu.VMEM((batch,), jnp.int32),
        pltpu.VMEM((batch,), jnp.int32),
        pltpu.VMEM((batch,), jnp.int32),
        pltpu.VMEM((batch, D), jnp.float32),
        pltpu.VMEM((batch, D), jnp.float32),
    ]
    if with_counts:
        scratch.append(pltpu.VMEM((batch, 16), jnp.float32))
    scratch.append(pltpu.VMEM((CH, D), jnp.float32))
    if with_counts:
        scratch.append(pltpu.VMEM((CH, 16), jnp.float32))
    scratch.append(pltpu.VMEM_SHARED((NPAD, D), jnp.float32))
    if with_counts:
        scratch.append(pltpu.VMEM_SHARED((NPAD, 16), jnp.float32))
    nsem = 10 if with_counts else 8
    scratch += [pltpu.SemaphoreType.DMA] * nsem

    @functools.partial(
        pl.kernel,
        mesh=_mesh,
        compiler_params=pltpu.CompilerParams(use_tc_tiling_on_sc=False),
        out_type=out_type,
        scratch_types=scratch,
    )
    def sc_agg(*refs):
        it = iter(refs)
        t_hbm = next(it)
        src_hbm = next(it)
        dst_hbm = next(it)
        zrow_hbm = next(it)
        zcnt_hbm = next(it) if with_counts else None
        ones_hbm = next(it) if with_counts else None
        pacc_hbm = next(it)
        pcnt_hbm = next(it) if with_counts else None
        SRC = (next(it), next(it))
        DST = (next(it), next(it))
        ROWS = (next(it), next(it))
        ones_v = next(it) if with_counts else None
        stage_v = next(it)
        stagec_v = next(it) if with_counts else None
        acc_s = next(it)
        cnt_s = next(it) if with_counts else None
        GS = (next(it), next(it))
        RS = (next(it), next(it))
        CS = (next(it), next(it)) if with_counts else None
        ISS = (next(it), next(it))
        IDS = (next(it), next(it))

        c = lax.axis_index("c")
        s = lax.axis_index("s")
        wid = s * NC + c

        def off(i):
            return pl.multiple_of(wid * EPWP + i * batch, 8)

        def idx_src(i, p):
            return pltpu.make_async_copy(
                src_hbm.at[pl.ds(off(i), batch)], SRC[p], ISS[p])

        def idx_dst(i, p):
            return pltpu.make_async_copy(
                dst_hbm.at[pl.ds(off(i), batch)], DST[p], IDS[p])

        def gath(p):
            return pltpu.make_async_copy(t_hbm.at[SRC[p]], ROWS[p], GS[p])

        def srow(p):
            return pltpu.make_async_copy(ROWS[p], acc_s.at[DST[p]], RS[p])

        def scnt(p):
            return pltpu.make_async_copy(ones_v, cnt_s.at[DST[p]], CS[p])

        pltpu.sync_copy(zrow_hbm, stage_v)
        if with_counts:
            pltpu.sync_copy(zcnt_hbm, stagec_v)
            pltpu.sync_copy(ones_hbm, ones_v)

        def zbody(j, carry):
            o = pl.multiple_of(s * RPT + j * CH, 8)
            pltpu.sync_copy(stage_v, acc_s.at[pl.ds(o, CH)])
            if with_counts:
                pltpu.sync_copy(stagec_v, cnt_s.at[pl.ds(o, CH)])
            return carry

        lax.fori_loop(0, NCH, zbody, 0)
        plsc.subcore_barrier()

        def step(i, p, wait_prev, next_gather, next2_idx):
            q = 1 - p
            gath(p).wait()
            idx_dst(i, p).wait()
            srow(p).start(add=True)
            if with_counts:
                scnt(p).start(add=True)
            if wait_prev:
                srow(q).wait()
                if with_counts:
                    scnt(q).wait()
            if next_gather:
                idx_src(i + 1, q).wait()
                gath(q).start()
                idx_dst(i + 1, q).start()
            if next2_idx:
                idx_src(i + 2, p).start()

        idx_src(0, 0).start()
        idx_dst(0, 0).start()
        idx_src(1, 1).start()
        idx_src(0, 0).wait()
        gath(0).start()

        step(0, 0, False, True, True)
        step(1, 1, True, True, True)

        def pair(k, carry):
            step(2 * k, 0, True, True, True)
            step(2 * k + 1, 1, True, True, True)
            return carry

        lax.fori_loop(1, niter // 2 - 1, pair, 0)
        step(niter - 2, 0, True, True, False)
        step(niter - 1, 1, True, False, False)
        srow(1).wait()
        if with_counts:
            scnt(1).wait()
        plsc.subcore_barrier()

        def wbody(j, carry):
            o = pl.multiple_of(s * RPT + j * CH, 8)
            pltpu.sync_copy(acc_s.at[pl.ds(o, CH)], stage_v)
            pltpu.sync_copy(stage_v, pacc_hbm.at[c, pl.ds(o, CH)])
            if with_counts:
                pltpu.sync_copy(cnt_s.at[pl.ds(o, CH)], stagec_v)
                pltpu.sync_copy(stagec_v, pcnt_hbm.at[c, pl.ds(o, CH)])
            return carry

        lax.fori_loop(0, NCH, wbody, 0)

    return sc_agg


_sc_agg_cnt = _make_sc_agg(80, True)
_sc_agg_nocnt = _make_sc_agg(128, False)


_BLK = 1000
_GRID = N // _BLK


def _dot_t(a, w):
    return lax.dot_general(a, w, (((1,), (1,)), ((), ())),
                           preferred_element_type=jnp.float32)


def _tc_in_body(x_ref, wl_ref, wr_ref, b_ref, t_ref, r_ref):
    xb = x_ref[...]
    t_ref[...] = _dot_t(xb, wl_ref[...])
    r_ref[...] = _dot_t(xb, wr_ref[...]) + b_ref[...]


def _tc_in(x, W_l, W_r, b):
    w_spec = pl.BlockSpec((D, D), lambda i: (0, 0))
    return pl.pallas_call(
        _tc_in_body,
        grid=(_GRID,),
        in_specs=[pl.BlockSpec((_BLK, D), lambda i: (i, 0)), w_spec, w_spec,
                  pl.BlockSpec((1, D), lambda i: (0, 0))],
        out_specs=[pl.BlockSpec((_BLK, D), lambda i: (i, 0))] * 2,
        out_shape=[jax.ShapeDtypeStruct((N, D), jnp.float32)] * 2,
    )(x, W_l, W_r, b.reshape(1, D))


def _tc_mid_body(pacc_ref, pcnt_ref, r_ref, wl_ref, wr_ref, b_ref,
                 t_ref, r2_ref):
    agg = pacc_ref[0] + pacc_ref[1]
    cnt = pcnt_ref[0, :, 0:1] + pcnt_ref[1, :, 0:1]
    mean = agg / jnp.maximum(cnt, 1.0)
    h = jnp.maximum(mean + r_ref[...], 0.0)
    t_ref[...] = _dot_t(h, wl_ref[...])
    r2_ref[...] = _dot_t(h, wr_ref[...]) + b_ref[...]


def _tc_mid(pacc, pcnt, r1, W_l, W_r, b):
    w_spec = pl.BlockSpec((D, D), lambda i: (0, 0))
    return pl.pallas_call(
        _tc_mid_body,
        grid=(_GRID,),
        in_specs=[pl.BlockSpec((NC, _BLK, D), lambda i: (0, i, 0)),
                  pl.BlockSpec((NC, _BLK, 16), lambda i: (0, i, 0)),
                  pl.BlockSpec((_BLK, D), lambda i: (i, 0)),
                  w_spec, w_spec,
                  pl.BlockSpec((1, D), lambda i: (0, 0))],
        out_specs=[pl.BlockSpec((_BLK, D), lambda i: (i, 0))] * 2,
        out_shape=[jax.ShapeDtypeStruct((N, D), jnp.float32)] * 2,
    )(pacc, pcnt, r1, W_l, W_r, b.reshape(1, D))


def _tc_out_body(pacc_ref, pcnt_ref, r_ref, o_ref):
    agg = pacc_ref[0] + pacc_ref[1]
    cnt = pcnt_ref[0, :, 0:1] + pcnt_ref[1, :, 0:1]
    o_ref[...] = agg / jnp.maximum(cnt, 1.0) + r_ref[...]


def _tc_out(pacc, pcnt, r2):
    return pl.pallas_call(
        _tc_out_body,
        grid=(_GRID,),
        in_specs=[pl.BlockSpec((NC, _BLK, D), lambda i: (0, i, 0)),
                  pl.BlockSpec((NC, _BLK, 16), lambda i: (0, i, 0)),
                  pl.BlockSpec((_BLK, D), lambda i: (i, 0))],
        out_specs=pl.BlockSpec((_BLK, D), lambda i: (i, 0)),
        out_shape=jax.ShapeDtypeStruct((N, D), jnp.float32),
    )(pacc, pcnt, r2)


def kernel(x, edge_index, W1_l, b1_l, W1_r, W2_l, b2_l, W2_r):
    src = edge_index[0].astype(jnp.int32)
    dst = edge_index[1].astype(jnp.int32)
    epw = E // NW
    pad = EPWP - epw
    src = jnp.concatenate(
        [src.reshape(NW, epw), jnp.zeros((NW, pad), jnp.int32)], axis=1
    ).reshape(-1)
    dst = jnp.concatenate(
        [dst.reshape(NW, epw),
         jnp.broadcast_to(jnp.arange(N, NPAD, dtype=jnp.int32), (NW, pad))],
        axis=1,
    ).reshape(-1)
    zrow = jnp.zeros((CH, D), jnp.float32)
    zcnt = jnp.zeros((CH, 16), jnp.float32)
    ones = jnp.ones((80, 16), jnp.float32)

    t1, r1 = _tc_in(x, W1_l, W1_r, b1_l)
    pacc1, pcnt1 = _sc_agg_cnt(t1, src, dst, zrow, zcnt, ones)
    t2, r2 = _tc_mid(pacc1, pcnt1, r1, W2_l, W2_r, b2_l)
    (pacc2,) = _sc_agg_nocnt(t2, src, dst, zrow)
    return _tc_out(pacc2, pcnt1, r2)


# speedup vs baseline: 2.6730x; 2.3590x over previous
import functools

import jax
import jax.numpy as jnp
from jax import lax
from jax.experimental import pallas as pl
from jax.experimental.pallas import tpu as pltpu
from jax.experimental.pallas import tpu_sc as plsc

N = 10000
E = 320000
D = 128

NC = 2
NS = 16
NW = NC * NS
NPAD = 10240
RPT = NPAD // NS
CH = 64
NCH = RPT // CH
EPW = E // NW

_mesh = plsc.VectorSubcoreMesh(core_axis_name="c", subcore_axis_name="s")


def _make_sc_agg(batch, with_counts):
    niter = EPW // batch
    assert niter >= 5

    out_type = [jax.ShapeDtypeStruct((NC, NPAD, D), jnp.float32)]
    if with_counts:
        out_type.append(jax.ShapeDtypeStruct((NC, NPAD, 16), jnp.float32))

    scratch = [
        pltpu.VMEM((batch,), jnp.int32),
        pltpu.VMEM((batch,), jnp.int32),
        pltpu.VMEM((batch,), jnp.int32),
        pltpu.VMEM((batch,), jnp.int32),
        pltpu.VMEM((batch, D), jnp.float32),
        pltpu.VMEM((batch, D), jnp.float32),
    ]
    if with_counts:
        scratch.append(pltpu.VMEM((batch, 16), jnp.float32))
    scratch.append(pltpu.VMEM((CH, D), jnp.float32))
    if with_counts:
        scratch.append(pltpu.VMEM((CH, 16), jnp.float32))
    scratch.append(pltpu.VMEM_SHARED((NPAD, D), jnp.float32))
    if with_counts:
        scratch.append(pltpu.VMEM_SHARED((NPAD, 16), jnp.float32))
    nsem = 10 if with_counts else 8
    scratch += [pltpu.SemaphoreType.DMA] * nsem

    @functools.partial(
        pl.kernel,
        mesh=_mesh,
        compiler_params=pltpu.CompilerParams(use_tc_tiling_on_sc=False),
        out_type=out_type,
        scratch_types=scratch,
    )
    def sc_agg(*refs):
        it = iter(refs)
        t_hbm = next(it)
        src_hbm = next(it)
        dst_hbm = next(it)
        zrow_hbm = next(it)
        zcnt_hbm = next(it) if with_counts else None
        ones_hbm = next(it) if with_counts else None
        pacc_hbm = next(it)
        pcnt_hbm = next(it) if with_counts else None
        SRC = (next(it), next(it))
        DST = (next(it), next(it))
        ROWS = (next(it), next(it))
        ones_v = next(it) if with_counts else None
        stage_v = next(it)
        stagec_v = next(it) if with_counts else None
        acc_s = next(it)
        cnt_s = next(it) if with_counts else None
        GS = (next(it), next(it))
        RS = (next(it), next(it))
        CS = (next(it), next(it)) if with_counts else None
        ISS = (next(it), next(it))
        IDS = (next(it), next(it))

        c = lax.axis_index("c")
        s = lax.axis_index("s")
        wid = s * NC + c

        def off(i):
            return pl.multiple_of(wid * EPW + i * batch, 8)

        def idx_src(i, p):
            return pltpu.make_async_copy(
                src_hbm.at[pl.ds(off(i), batch)], SRC[p], ISS[p])

        def idx_dst(i, p):
            return pltpu.make_async_copy(
                dst_hbm.at[pl.ds(off(i), batch)], DST[p], IDS[p])

        def gath(p):
            return pltpu.make_async_copy(t_hbm.at[SRC[p]], ROWS[p], GS[p])

        def srow(p):
            return pltpu.make_async_copy(ROWS[p], acc_s.at[DST[p]], RS[p])

        def scnt(p):
            return pltpu.make_async_copy(ones_v, cnt_s.at[DST[p]], CS[p])

        pltpu.sync_copy(zrow_hbm, stage_v)
        if with_counts:
            pltpu.sync_copy(zcnt_hbm, stagec_v)
            pltpu.sync_copy(ones_hbm, ones_v)

        def zbody(j, carry):
            o = pl.multiple_of(s * RPT + j * CH, 8)
            pltpu.sync_copy(stage_v, acc_s.at[pl.ds(o, CH)])
            if with_counts:
                pltpu.sync_copy(stagec_v, cnt_s.at[pl.ds(o, CH)])
            return carry

        lax.fori_loop(0, NCH, zbody, 0)
        plsc.subcore_barrier()

        def step(i, p, wait_prev, next_gather, next2_idx):
            q = 1 - p
            gath(p).wait()
            idx_dst(i, p).wait()
            srow(p).start(add=True)
            if with_counts:
                scnt(p).start(add=True)
            if wait_prev:
                srow(q).wait()
                if with_counts:
                    scnt(q).wait()
            if next_gather:
                idx_src(i + 1, q).wait()
                gath(q).start()
                idx_dst(i + 1, q).start()
            if next2_idx:
                idx_src(i + 2, p).start()

        idx_src(0, 0).start()
        idx_dst(0, 0).start()
        idx_src(1, 1).start()
        idx_src(0, 0).wait()
        gath(0).start()

        step(0, 0, False, True, True)
        step(1, 1, True, True, True)

        def pair(k, carry):
            step(2 * k, 0, True, True, True)
            step(2 * k + 1, 1, True, True, True)
            return carry

        if niter % 2 == 0:
            lax.fori_loop(1, (niter - 2) // 2, pair, 0)
        else:
            lax.fori_loop(1, (niter - 3) // 2, pair, 0)
            step(niter - 3, 0, True, True, True)
        step(niter - 2, (niter - 2) % 2, True, True, False)
        step(niter - 1, (niter - 1) % 2, True, False, False)
        srow((niter - 1) % 2).wait()
        if with_counts:
            scnt((niter - 1) % 2).wait()
        plsc.subcore_barrier()

        def wbody(j, carry):
            o = pl.multiple_of(s * RPT + j * CH, 8)
            pltpu.sync_copy(acc_s.at[pl.ds(o, CH)], stage_v)
            pltpu.sync_copy(stage_v, pacc_hbm.at[c, pl.ds(o, CH)])
            if with_counts:
                pltpu.sync_copy(cnt_s.at[pl.ds(o, CH)], stagec_v)
                pltpu.sync_copy(stagec_v, pcnt_hbm.at[c, pl.ds(o, CH)])
            return carry

        lax.fori_loop(0, NCH, wbody, 0)

    return sc_agg


_sc_agg_cnt = _make_sc_agg(80, True)
_sc_agg_nocnt = _make_sc_agg(80, False)


_BLK = 1000
_GRID = N // _BLK


def _dot_t(a, w):
    return lax.dot_general(a, w, (((1,), (1,)), ((), ())),
                           preferred_element_type=jnp.float32)


def _tc_in_body(x_ref, wl_ref, wr_ref, b_ref, t_ref, r_ref):
    xb = x_ref[...]
    t_ref[...] = _dot_t(xb, wl_ref[...])
    r_ref[...] = _dot_t(xb, wr_ref[...]) + b_ref[...]


def _tc_in(x, W_l, W_r, b):
    w_spec = pl.BlockSpec((D, D), lambda i: (0, 0))
    return pl.pallas_call(
        _tc_in_body,
        grid=(_GRID,),
        in_specs=[pl.BlockSpec((_BLK, D), lambda i: (i, 0)), w_spec, w_spec,
                  pl.BlockSpec((1, D), lambda i: (0, 0))],
        out_specs=[pl.BlockSpec((_BLK, D), lambda i: (i, 0))] * 2,
        out_shape=[jax.ShapeDtypeStruct((N, D), jnp.float32)] * 2,
    )(x, W_l, W_r, b.reshape(1, D))


def _tc_mid_body(pacc_ref, pcnt_ref, r_ref, wl_ref, wr_ref, b_ref,
                 t_ref, r2_ref):
    agg = pacc_ref[0] + pacc_ref[1]
    cnt = pcnt_ref[0, :, 0:1] + pcnt_ref[1, :, 0:1]
    mean = agg / jnp.maximum(cnt, 1.0)
    h = jnp.maximum(mean + r_ref[...], 0.0)
    t_ref[...] = _dot_t(h, wl_ref[...])
    r2_ref[...] = _dot_t(h, wr_ref[...]) + b_ref[...]


def _tc_mid(pacc, pcnt, r1, W_l, W_r, b):
    w_spec = pl.BlockSpec((D, D), lambda i: (0, 0))
    return pl.pallas_call(
        _tc_mid_body,
        grid=(_GRID,),
        in_specs=[pl.BlockSpec((NC, _BLK, D), lambda i: (0, i, 0)),
                  pl.BlockSpec((NC, _BLK, 16), lambda i: (0, i, 0)),
                  pl.BlockSpec((_BLK, D), lambda i: (i, 0)),
                  w_spec, w_spec,
                  pl.BlockSpec((1, D), lambda i: (0, 0))],
        out_specs=[pl.BlockSpec((_BLK, D), lambda i: (i, 0))] * 2,
        out_shape=[jax.ShapeDtypeStruct((N, D), jnp.float32)] * 2,
    )(pacc, pcnt, r1, W_l, W_r, b.reshape(1, D))


def _tc_out_body(pacc_ref, pcnt_ref, r_ref, o_ref):
    agg = pacc_ref[0] + pacc_ref[1]
    cnt = pcnt_ref[0, :, 0:1] + pcnt_ref[1, :, 0:1]
    o_ref[...] = agg / jnp.maximum(cnt, 1.0) + r_ref[...]


def _tc_out(pacc, pcnt, r2):
    return pl.pallas_call(
        _tc_out_body,
        grid=(_GRID,),
        in_specs=[pl.BlockSpec((NC, _BLK, D), lambda i: (0, i, 0)),
                  pl.BlockSpec((NC, _BLK, 16), lambda i: (0, i, 0)),
                  pl.BlockSpec((_BLK, D), lambda i: (i, 0))],
        out_specs=pl.BlockSpec((_BLK, D), lambda i: (i, 0)),
        out_shape=jax.ShapeDtypeStruct((N, D), jnp.float32),
    )(pacc, pcnt, r2)


def kernel(x, edge_index, W1_l, b1_l, W1_r, W2_l, b2_l, W2_r):
    src = edge_index[0].astype(jnp.int32)
    dst = edge_index[1].astype(jnp.int32)
    zrow = jnp.zeros((CH, D), jnp.float32)
    zcnt = jnp.zeros((CH, 16), jnp.float32)
    ones = jnp.ones((80, 16), jnp.float32)

    t1, r1 = _tc_in(x, W1_l, W1_r, b1_l)
    pacc1, pcnt1 = _sc_agg_cnt(t1, src, dst, zrow, zcnt, ones)
    t2, r2 = _tc_mid(pacc1, pcnt1, r1, W2_l, W2_r, b2_l)
    (pacc2,) = _sc_agg_nocnt(t2, src, dst, zrow)
    return _tc_out(pacc2, pcnt1, r2)


# speedup vs baseline: 2.7794x; 1.0398x over previous
import functools

import jax
import jax.numpy as jnp
from jax import lax
from jax.experimental import pallas as pl
from jax.experimental.pallas import tpu as pltpu
from jax.experimental.pallas import tpu_sc as plsc

N = 10000
E = 320000
D = 128

NC = 2
NS = 16
NW = NC * NS
NPAD = 10240
RPT = NPAD // NS
CH = 64
NCH = RPT // CH
EPW = E // NW

_mesh = plsc.VectorSubcoreMesh(core_axis_name="c", subcore_axis_name="s")


def _make_sc_agg(batch, with_counts):
    niter = EPW // batch
    assert niter >= 5

    out_type = [jax.ShapeDtypeStruct((NC, NPAD, D), jnp.bfloat16)]
    if with_counts:
        out_type.append(jax.ShapeDtypeStruct((NC, NPAD, 16), jnp.float32))

    scratch = [
        pltpu.VMEM((batch,), jnp.int32),
        pltpu.VMEM((batch,), jnp.int32),
        pltpu.VMEM((batch,), jnp.int32),
        pltpu.VMEM((batch,), jnp.int32),
        pltpu.VMEM((batch, D), jnp.bfloat16),
        pltpu.VMEM((batch, D), jnp.bfloat16),
    ]
    if with_counts:
        scratch.append(pltpu.VMEM((batch, 16), jnp.float32))
    scratch.append(pltpu.VMEM((CH, D), jnp.bfloat16))
    if with_counts:
        scratch.append(pltpu.VMEM((CH, 16), jnp.float32))
    scratch.append(pltpu.VMEM_SHARED((NPAD, D), jnp.bfloat16))
    if with_counts:
        scratch.append(pltpu.VMEM_SHARED((NPAD, 16), jnp.float32))
    nsem = 10 if with_counts else 8
    scratch += [pltpu.SemaphoreType.DMA] * nsem

    @functools.partial(
        pl.kernel,
        mesh=_mesh,
        compiler_params=pltpu.CompilerParams(use_tc_tiling_on_sc=False),
        out_type=out_type,
        scratch_types=scratch,
    )
    def sc_agg(*refs):
        it = iter(refs)
        t_hbm = next(it)
        src_hbm = next(it)
        dst_hbm = next(it)
        zrow_hbm = next(it)
        zcnt_hbm = next(it) if with_counts else None
        ones_hbm = next(it) if with_counts else None
        pacc_hbm = next(it)
        pcnt_hbm = next(it) if with_counts else None
        SRC = (next(it), next(it))
        DST = (next(it), next(it))
        ROWS = (next(it), next(it))
        ones_v = next(it) if with_counts else None
        stage_v = next(it)
        stagec_v = next(it) if with_counts else None
        acc_s = next(it)
        cnt_s = next(it) if with_counts else None
        GS = (next(it), next(it))
        RS = (next(it), next(it))
        CS = (next(it), next(it)) if with_counts else None
        ISS = (next(it), next(it))
        IDS = (next(it), next(it))

        c = lax.axis_index("c")
        s = lax.axis_index("s")
        wid = s * NC + c

        def off(i):
            return pl.multiple_of(wid * EPW + i * batch, 8)

        def idx_src(i, p):
            return pltpu.make_async_copy(
                src_hbm.at[pl.ds(off(i), batch)], SRC[p], ISS[p])

        def idx_dst(i, p):
            return pltpu.make_async_copy(
                dst_hbm.at[pl.ds(off(i), batch)], DST[p], IDS[p])

        def gath(p):
            return pltpu.make_async_copy(t_hbm.at[SRC[p]], ROWS[p], GS[p])

        def srow(p):
            return pltpu.make_async_copy(ROWS[p], acc_s.at[DST[p]], RS[p])

        def scnt(p):
            return pltpu.make_async_copy(ones_v, cnt_s.at[DST[p]], CS[p])

        pltpu.sync_copy(zrow_hbm, stage_v)
        if with_counts:
            pltpu.sync_copy(zcnt_hbm, stagec_v)
            pltpu.sync_copy(ones_hbm, ones_v)

        def zbody(j, carry):
            o = pl.multiple_of(s * RPT + j * CH, 8)
            pltpu.sync_copy(stage_v, acc_s.at[pl.ds(o, CH)])
            if with_counts:
                pltpu.sync_copy(stagec_v, cnt_s.at[pl.ds(o, CH)])
            return carry

        lax.fori_loop(0, NCH, zbody, 0)
        plsc.subcore_barrier()

        def step(i, p, wait_prev, next_gather, next2_idx):
            q = 1 - p
            gath(p).wait()
            idx_dst(i, p).wait()
            srow(p).start(add=True)
            if with_counts:
                scnt(p).start(add=True)
            if wait_prev:
                srow(q).wait()
                if with_counts:
                    scnt(q).wait()
            if next_gather:
                idx_src(i + 1, q).wait()
                gath(q).start()
                idx_dst(i + 1, q).start()
            if next2_idx:
                idx_src(i + 2, p).start()

        idx_src(0, 0).start()
        idx_dst(0, 0).start()
        idx_src(1, 1).start()
        idx_src(0, 0).wait()
        gath(0).start()

        step(0, 0, False, True, True)
        step(1, 1, True, True, True)

        def pair(k, carry):
            step(2 * k, 0, True, True, True)
            step(2 * k + 1, 1, True, True, True)
            return carry

        if niter % 2 == 0:
            lax.fori_loop(1, (niter - 2) // 2, pair, 0)
        else:
            lax.fori_loop(1, (niter - 3) // 2, pair, 0)
            step(niter - 3, 0, True, True, True)
        step(niter - 2, (niter - 2) % 2, True, True, False)
        step(niter - 1, (niter - 1) % 2, True, False, False)
        srow((niter - 1) % 2).wait()
        if with_counts:
            scnt((niter - 1) % 2).wait()
        plsc.subcore_barrier()

        def wbody(j, carry):
            o = pl.multiple_of(s * RPT + j * CH, 8)
            pltpu.sync_copy(acc_s.at[pl.ds(o, CH)], stage_v)
            pltpu.sync_copy(stage_v, pacc_hbm.at[c, pl.ds(o, CH)])
            if with_counts:
                pltpu.sync_copy(cnt_s.at[pl.ds(o, CH)], stagec_v)
                pltpu.sync_copy(stagec_v, pcnt_hbm.at[c, pl.ds(o, CH)])
            return carry

        lax.fori_loop(0, NCH, wbody, 0)

    return sc_agg


_sc_agg_cnt = _make_sc_agg(80, True)
_sc_agg_nocnt = _make_sc_agg(80, False)


_BLK = 1024
_GRID = NPAD // _BLK


def _dot_t(a, w):
    return lax.dot_general(a, w, (((1,), (1,)), ((), ())),
                           preferred_element_type=jnp.float32)


def _tc_in_body(x_ref, wl_ref, wr_ref, b_ref, t_ref, r_ref):
    xb = x_ref[...]
    t_ref[...] = _dot_t(xb, wl_ref[...]).astype(jnp.bfloat16)
    r_ref[...] = _dot_t(xb, wr_ref[...]) + b_ref[...]


def _tc_in(x, W_l, W_r, b):
    w_spec = pl.BlockSpec((D, D), lambda i: (0, 0))
    return pl.pallas_call(
        _tc_in_body,
        grid=(_GRID,),
        in_specs=[pl.BlockSpec((_BLK, D), lambda i: (i, 0)), w_spec, w_spec,
                  pl.BlockSpec((1, D), lambda i: (0, 0))],
        out_specs=[pl.BlockSpec((_BLK, D), lambda i: (i, 0))] * 2,
        out_shape=[jax.ShapeDtypeStruct((NPAD, D), jnp.bfloat16),
                   jax.ShapeDtypeStruct((NPAD, D), jnp.float32)],
    )(x, W_l, W_r, b.reshape(1, D))


def _tc_mid_body(pacc_ref, pcnt_ref, r_ref, wl_ref, wr_ref, b_ref,
                 t_ref, r2_ref):
    agg = pacc_ref[0].astype(jnp.float32) + pacc_ref[1].astype(jnp.float32)
    cnt = pcnt_ref[0, :, 0:1] + pcnt_ref[1, :, 0:1]
    mean = agg / jnp.maximum(cnt, 1.0)
    h = jnp.maximum(mean + r_ref[...], 0.0)
    t_ref[...] = _dot_t(h, wl_ref[...]).astype(jnp.bfloat16)
    r2_ref[...] = _dot_t(h, wr_ref[...]) + b_ref[...]


def _tc_mid(pacc, pcnt, r1, W_l, W_r, b):
    w_spec = pl.BlockSpec((D, D), lambda i: (0, 0))
    return pl.pallas_call(
        _tc_mid_body,
        grid=(_GRID,),
        in_specs=[pl.BlockSpec((NC, _BLK, D), lambda i: (0, i, 0)),
                  pl.BlockSpec((NC, _BLK, 16), lambda i: (0, i, 0)),
                  pl.BlockSpec((_BLK, D), lambda i: (i, 0)),
                  w_spec, w_spec,
                  pl.BlockSpec((1, D), lambda i: (0, 0))],
        out_specs=[pl.BlockSpec((_BLK, D), lambda i: (i, 0))] * 2,
        out_shape=[jax.ShapeDtypeStruct((NPAD, D), jnp.bfloat16),
                   jax.ShapeDtypeStruct((NPAD, D), jnp.float32)],
    )(pacc, pcnt, r1, W_l, W_r, b.reshape(1, D))


def _tc_out_body(pacc_ref, pcnt_ref, r_ref, o_ref):
    agg = pacc_ref[0].astype(jnp.float32) + pacc_ref[1].astype(jnp.float32)
    cnt = pcnt_ref[0, :, 0:1] + pcnt_ref[1, :, 0:1]
    o_ref[...] = agg / jnp.maximum(cnt, 1.0) + r_ref[...]


def _tc_out(pacc, pcnt, r2):
    return pl.pallas_call(
        _tc_out_body,
        grid=(_GRID,),
        in_specs=[pl.BlockSpec((NC, _BLK, D), lambda i: (0, i, 0)),
                  pl.BlockSpec((NC, _BLK, 16), lambda i: (0, i, 0)),
                  pl.BlockSpec((_BLK, D), lambda i: (i, 0))],
        out_specs=pl.BlockSpec((_BLK, D), lambda i: (i, 0)),
        out_shape=jax.ShapeDtypeStruct((NPAD, D), jnp.float32),
    )(pacc, pcnt, r2)


def kernel(x, edge_index, W1_l, b1_l, W1_r, W2_l, b2_l, W2_r):
    src = edge_index[0].astype(jnp.int32)
    dst = edge_index[1].astype(jnp.int32)
    x = jnp.pad(x, ((0, NPAD - N), (0, 0)))
    zrow = jnp.zeros((CH, D), jnp.bfloat16)
    zcnt = jnp.zeros((CH, 16), jnp.float32)
    ones = jnp.ones((80, 16), jnp.float32)

    t1, r1 = _tc_in(x, W1_l, W1_r, b1_l)
    pacc1, pcnt1 = _sc_agg_cnt(t1, src, dst, zrow, zcnt, ones)
    t2, r2 = _tc_mid(pacc1, pcnt1, r1, W2_l, W2_r, b2_l)
    (pacc2,) = _sc_agg_nocnt(t2, src, dst, zrow)
    return _tc_out(pacc2, pcnt1, r2)[:N]


# speedup vs baseline: 3.5958x; 1.2937x over previous
import functools

import jax
import jax.numpy as jnp
from jax import lax
from jax.experimental import pallas as pl
from jax.experimental.pallas import tpu as pltpu
from jax.experimental.pallas import tpu_sc as plsc

N = 10000
E = 320000
D = 128

NC = 2
NS = 16
NW = NC * NS
NPAD = 10240
RPT = NPAD // NS
CH = 64
NCH = RPT // CH
EPW = E // NW

_mesh = plsc.VectorSubcoreMesh(core_axis_name="c", subcore_axis_name="s")


def _make_sc_agg(batch, with_counts):
    niter = EPW // batch

    out_type = [jax.ShapeDtypeStruct((NC, NPAD, D), jnp.bfloat16)]
    if with_counts:
        out_type.append(jax.ShapeDtypeStruct((NC, NPAD, 16), jnp.float32))

    scratch = (
        [pltpu.VMEM((batch,), jnp.int32)] * 3
        + [pltpu.VMEM((batch,), jnp.int32)] * 3
        + [pltpu.VMEM((batch, D), jnp.bfloat16)] * 3
    )
    if with_counts:
        scratch.append(pltpu.VMEM((batch, 16), jnp.float32))
    scratch.append(pltpu.VMEM((CH, D), jnp.bfloat16))
    if with_counts:
        scratch.append(pltpu.VMEM((CH, 16), jnp.float32))
    scratch.append(pltpu.VMEM_SHARED((NPAD, D), jnp.bfloat16))
    if with_counts:
        scratch.append(pltpu.VMEM_SHARED((NPAD, 16), jnp.float32))
    nsem = 15 if with_counts else 12
    scratch += [pltpu.SemaphoreType.DMA] * nsem

    @functools.partial(
        pl.kernel,
        mesh=_mesh,
        compiler_params=pltpu.CompilerParams(use_tc_tiling_on_sc=False),
        out_type=out_type,
        scratch_types=scratch,
    )
    def sc_agg(*refs):
        it = iter(refs)
        t_hbm = next(it)
        src_hbm = next(it)
        dst_hbm = next(it)
        zrow_hbm = next(it)
        zcnt_hbm = next(it) if with_counts else None
        ones_hbm = next(it) if with_counts else None
        pacc_hbm = next(it)
        pcnt_hbm = next(it) if with_counts else None
        SRC = (next(it), next(it), next(it))
        DST = (next(it), next(it), next(it))
        ROWS = (next(it), next(it), next(it))
        ones_v = next(it) if with_counts else None
        stage_v = next(it)
        stagec_v = next(it) if with_counts else None
        acc_s = next(it)
        cnt_s = next(it) if with_counts else None
        GS = (next(it), next(it), next(it))
        RS = (next(it), next(it), next(it))
        CS = (next(it), next(it), next(it)) if with_counts else None
        ISS = (next(it), next(it), next(it))
        IDS = (next(it), next(it), next(it))

        c = lax.axis_index("c")
        s = lax.axis_index("s")
        wid = s * NC + c

        def off(i):
            return pl.multiple_of(wid * EPW + i * batch, 8)

        def idx_src(i, p):
            return pltpu.make_async_copy(
                src_hbm.at[pl.ds(off(i), batch)], SRC[p], ISS[p])

        def idx_dst(i, p):
            return pltpu.make_async_copy(
                dst_hbm.at[pl.ds(off(i), batch)], DST[p], IDS[p])

        def gath(p):
            return pltpu.make_async_copy(t_hbm.at[SRC[p]], ROWS[p], GS[p])

        def srow(p):
            return pltpu.make_async_copy(ROWS[p], acc_s.at[DST[p]], RS[p])

        def scnt(p):
            return pltpu.make_async_copy(ones_v, cnt_s.at[DST[p]], CS[p])

        pltpu.sync_copy(zrow_hbm, stage_v)
        if with_counts:
            pltpu.sync_copy(zcnt_hbm, stagec_v)
            pltpu.sync_copy(ones_hbm, ones_v)

        def zbody(j, carry):
            o = pl.multiple_of(s * RPT + j * CH, 8)
            pltpu.sync_copy(stage_v, acc_s.at[pl.ds(o, CH)])
            if with_counts:
                pltpu.sync_copy(stagec_v, cnt_s.at[pl.ds(o, CH)])
            return carry

        lax.fori_loop(0, NCH, zbody, 0)
        plsc.subcore_barrier()

        def step(i, p, wait_prev, ahead2, ahead3):
            pm1 = (p + 2) % 3
            gath(p).wait()
            idx_dst(i, p).wait()
            srow(p).start(add=True)
            if with_counts:
                scnt(p).start(add=True)
            if wait_prev:
                srow(pm1).wait()
                if with_counts:
                    scnt(pm1).wait()
            if ahead2:
                idx_src(i + 2, pm1).wait()
                gath(pm1).start()
                idx_dst(i + 2, pm1).start()
            if ahead3:
                idx_src(i + 3, p).start()

        idx_src(0, 0).start()
        idx_dst(0, 0).start()
        idx_src(1, 1).start()
        idx_dst(1, 1).start()
        idx_src(2, 2).start()
        idx_src(0, 0).wait()
        gath(0).start()
        idx_src(1, 1).wait()
        gath(1).start()

        step(0, 0, False, True, True)
        step(1, 1, True, True, True)

        def triple(k, carry):
            i = 3 * k + 2
            step(i, 2, True, True, True)
            step(i + 1, 0, True, True, True)
            step(i + 2, 1, True, True, True)
            return carry

        assert niter % 2 == 1 and (niter - 2) % 3 == 0
        lax.fori_loop(0, (niter - 2) // 3 - 1, triple, 0)
        step(niter - 3, (niter - 3) % 3, True, True, False)
        step(niter - 2, (niter - 2) % 3, True, False, False)
        step(niter - 1, (niter - 1) % 3, True, False, False)
        srow((niter - 1) % 3).wait()
        if with_counts:
            scnt((niter - 1) % 3).wait()
        plsc.subcore_barrier()

        def wbody(j, carry):
            o = pl.multiple_of(s * RPT + j * CH, 8)
            pltpu.sync_copy(acc_s.at[pl.ds(o, CH)], stage_v)
            pltpu.sync_copy(stage_v, pacc_hbm.at[c, pl.ds(o, CH)])
            if with_counts:
                pltpu.sync_copy(cnt_s.at[pl.ds(o, CH)], stagec_v)
                pltpu.sync_copy(stagec_v, pcnt_hbm.at[c, pl.ds(o, CH)])
            return carry

        lax.fori_loop(0, NCH, wbody, 0)

    return sc_agg


_sc_agg_cnt = _make_sc_agg(80, True)
_sc_agg_nocnt = _make_sc_agg(80, False)


_BLK = 1024
_GRID = NPAD // _BLK


def _dot_t(a, w):
    return lax.dot_general(a, w, (((1,), (1,)), ((), ())),
                           preferred_element_type=jnp.float32)


def _tc_in_body(x_ref, wl_ref, wr_ref, b_ref, t_ref, r_ref):
    xb = x_ref[...]
    t_ref[...] = _dot_t(xb, wl_ref[...]).astype(jnp.bfloat16)
    r_ref[...] = _dot_t(xb, wr_ref[...]) + b_ref[...]


def _tc_in(x, W_l, W_r, b):
    w_spec = pl.BlockSpec((D, D), lambda i: (0, 0))
    return pl.pallas_call(
        _tc_in_body,
        grid=(_GRID,),
        in_specs=[pl.BlockSpec((_BLK, D), lambda i: (i, 0)), w_spec, w_spec,
                  pl.BlockSpec((1, D), lambda i: (0, 0))],
        out_specs=[pl.BlockSpec((_BLK, D), lambda i: (i, 0))] * 2,
        out_shape=[jax.ShapeDtypeStruct((NPAD, D), jnp.bfloat16),
                   jax.ShapeDtypeStruct((NPAD, D), jnp.float32)],
    )(x, W_l, W_r, b.reshape(1, D))


def _tc_mid_body(pacc_ref, pcnt_ref, r_ref, wl_ref, wr_ref, b_ref,
                 t_ref, r2_ref):
    agg = pacc_ref[0].astype(jnp.float32) + pacc_ref[1].astype(jnp.float32)
    cnt = pcnt_ref[0, :, 0:1] + pcnt_ref[1, :, 0:1]
    mean = agg / jnp.maximum(cnt, 1.0)
    h = jnp.maximum(mean + r_ref[...], 0.0)
    t_ref[...] = _dot_t(h, wl_ref[...]).astype(jnp.bfloat16)
    r2_ref[...] = _dot_t(h, wr_ref[...]) + b_ref[...]


def _tc_mid(pacc, pcnt, r1, W_l, W_r, b):
    w_spec = pl.BlockSpec((D, D), lambda i: (0, 0))
    return pl.pallas_call(
        _tc_mid_body,
        grid=(_GRID,),
        in_specs=[pl.BlockSpec((NC, _BLK, D), lambda i: (0, i, 0)),
                  pl.BlockSpec((NC, _BLK, 16), lambda i: (0, i, 0)),
                  pl.BlockSpec((_BLK, D), lambda i: (i, 0)),
                  w_spec, w_spec,
                  pl.BlockSpec((1, D), lambda i: (0, 0))],
        out_specs=[pl.BlockSpec((_BLK, D), lambda i: (i, 0))] * 2,
        out_shape=[jax.ShapeDtypeStruct((NPAD, D), jnp.bfloat16),
                   jax.ShapeDtypeStruct((NPAD, D), jnp.float32)],
    )(pacc, pcnt, r1, W_l, W_r, b.reshape(1, D))


def _tc_out_body(pacc_ref, pcnt_ref, r_ref, o_ref):
    agg = pacc_ref[0].astype(jnp.float32) + pacc_ref[1].astype(jnp.float32)
    cnt = pcnt_ref[0, :, 0:1] + pcnt_ref[1, :, 0:1]
    o_ref[...] = agg / jnp.maximum(cnt, 1.0) + r_ref[...]


def _tc_out(pacc, pcnt, r2):
    return pl.pallas_call(
        _tc_out_body,
        grid=(_GRID,),
        in_specs=[pl.BlockSpec((NC, _BLK, D), lambda i: (0, i, 0)),
                  pl.BlockSpec((NC, _BLK, 16), lambda i: (0, i, 0)),
                  pl.BlockSpec((_BLK, D), lambda i: (i, 0))],
        out_specs=pl.BlockSpec((_BLK, D), lambda i: (i, 0)),
        out_shape=jax.ShapeDtypeStruct((NPAD, D), jnp.float32),
    )(pacc, pcnt, r2)


def kernel(x, edge_index, W1_l, b1_l, W1_r, W2_l, b2_l, W2_r):
    src = edge_index[0].astype(jnp.int32)
    dst = edge_index[1].astype(jnp.int32)
    x = jnp.pad(x, ((0, NPAD - N), (0, 0)))
    zrow = jnp.zeros((CH, D), jnp.bfloat16)
    zcnt = jnp.zeros((CH, 16), jnp.float32)
    ones = jnp.ones((80, 16), jnp.float32)

    t1, r1 = _tc_in(x, W1_l, W1_r, b1_l)
    pacc1, pcnt1 = _sc_agg_cnt(t1, src, dst, zrow, zcnt, ones)
    t2, r2 = _tc_mid(pacc1, pcnt1, r1, W2_l, W2_r, b2_l)
    (pacc2,) = _sc_agg_nocnt(t2, src, dst, zrow)
    return _tc_out(pacc2, pcnt1, r2)[:N]
